# baseline (speedup 1.0000x reference)
import jax
import jax.numpy as jnp
from jax import lax
from jax.experimental import pallas as pl
from jax.experimental.pallas import tpu as pltpu

_EPS = 1e-8
_NBUF = 8
_S_CHUNK = 288


def _make_body(B, S, D, s_chunk):
    n_s = S // s_chunk
    n_units = B * n_s
    inv_sqrt_d = 1.0 / (D ** 0.5)

    def body(idx_ref, h_hbm, pe_ref, w_ref, o_hbm,
             in_buf, out_buf, in_sems, out_sems):
        def in_copy(u, slot):
            b = u // n_s
            s = lax.rem(u, n_s)
            return pltpu.make_async_copy(
                h_hbm.at[b, pl.ds(s * s_chunk, s_chunk), :],
                in_buf.at[slot],
                in_sems.at[slot],
            )

        def out_copy(u, slot):
            b = u // n_s
            s = lax.rem(u, n_s)
            return pltpu.make_async_copy(
                out_buf.at[slot],
                o_hbm.at[b, pl.ds(s * s_chunk, s_chunk), :],
                out_sems.at[slot],
            )

        for i in range(_NBUF):
            in_copy(i, i).start()

        def step(u, carry):
            slot = lax.rem(u, _NBUF)
            in_copy(u, slot).wait()

            @pl.when(u >= _NBUF)
            def _():
                out_copy(u - _NBUF, slot).wait()

            b = u // n_s
            pe_row = pe_ref[idx_ref[b], 0, :]
            x = in_buf[slot] + pe_row[None, :]
            ssq = jnp.sum(x * x, axis=-1, keepdims=True)
            recip = 1.0 / (jnp.sqrt(ssq) * inv_sqrt_d + _EPS)
            out_buf[slot] = x * (recip * w_ref[...])

            out_copy(u, slot).start()

            @pl.when(u + _NBUF < n_units)
            def _():
                in_copy(u + _NBUF, slot).start()

            return carry

        lax.fori_loop(0, n_units, step, 0)

        for i in range(_NBUF):
            u = n_units - _NBUF + i
            out_copy(u, u % _NBUF).wait()

    return body


def kernel(hidden_state, index, pos_embed, weight):
    B, S, D = hidden_state.shape
    idx = index.astype(jnp.int32)
    w2d = weight.reshape(1, D)
    s_chunk = _S_CHUNK if S % _S_CHUNK == 0 else S

    grid_spec = pltpu.PrefetchScalarGridSpec(
        num_scalar_prefetch=1,
        grid=(1,),
        in_specs=[
            pl.BlockSpec(memory_space=pl.ANY),
            pl.BlockSpec((pos_embed.shape[0], 1, D), lambda i, idx_ref: (0, 0, 0)),
            pl.BlockSpec((1, D), lambda i, idx_ref: (0, 0)),
        ],
        out_specs=pl.BlockSpec(memory_space=pl.ANY),
        scratch_shapes=[
            pltpu.VMEM((_NBUF, s_chunk, D), jnp.float32),
            pltpu.VMEM((_NBUF, s_chunk, D), jnp.float32),
            pltpu.SemaphoreType.DMA((_NBUF,)),
            pltpu.SemaphoreType.DMA((_NBUF,)),
        ],
    )
    return pl.pallas_call(
        _make_body(B, S, D, s_chunk),
        grid_spec=grid_spec,
        out_shape=jax.ShapeDtypeStruct((B, S, D), jnp.float32),
    )(idx, hidden_state, pos_embed, w2d)
